# baseline (speedup 1.0000x reference)
import functools

import jax
import jax.numpy as jnp
from jax import lax
from jax.experimental import pallas as pl
from jax.experimental.pallas import tpu as pltpu

_B, _N, _K = 128, 32768, 100
_R = 8


def _topk_body(pref_ref, x0_ref, x1_ref, x2_ref, m_ref,
               o0_ref, o1_ref, o2_ref, o3_ref, ws_ref):
    p0, p1, p2 = pref_ref[0], pref_ref[1], pref_ref[2]
    x0 = x0_ref[...]
    x1 = x1_ref[...]
    x2 = x2_ref[...]
    ws_ref[...] = (p0 * x0 + p1 * x1) + p2 * x2
    iota = lax.broadcasted_iota(jnp.int32, (_R, _N), 1)
    lane = lax.broadcasted_iota(jnp.int32, (_R, 128), 1)

    def body(s, carry):
        a0, a1, a2, a3 = carry
        w = ws_ref[...]
        mval = jnp.min(w, axis=1, keepdims=True)
        idx = jnp.min(jnp.where(w == mval, iota, _N), axis=1, keepdims=True)
        onehot = iota == idx
        v0 = jnp.sum(jnp.where(onehot, x0, 0.0), axis=1, keepdims=True)
        v1 = jnp.sum(jnp.where(onehot, x1, 0.0), axis=1, keepdims=True)
        v2 = jnp.sum(jnp.where(onehot, x2, 0.0), axis=1, keepdims=True)
        v3 = jnp.sum(jnp.where(onehot, m_ref[...], 0.0), axis=1, keepdims=True)
        hit = lane == s
        a0 = jnp.where(hit, v0, a0)
        a1 = jnp.where(hit, v1, a1)
        a2 = jnp.where(hit, v2, a2)
        a3 = jnp.where(hit, v3, a3)
        ws_ref[...] = jnp.where(onehot, jnp.inf, w)
        return a0, a1, a2, a3

    z = jnp.zeros((_R, 128), jnp.float32)
    a0, a1, a2, a3 = lax.fori_loop(0, _K, body, (z, z, z, z))
    o0_ref[...] = a0[:, :_K]
    o1_ref[...] = a1[:, :_K]
    o2_ref[...] = a2[:, :_K]
    o3_ref[...] = a3[:, :_K]


@jax.jit
def _run(sols, sols_mask, pref):
    x0 = sols[:, :, 0]
    x1 = sols[:, :, 1]
    x2 = sols[:, :, 2]
    row_spec = pl.BlockSpec((_R, _N), lambda i: (i, 0))
    out_spec = pl.BlockSpec((_R, _K), lambda i: (i, 0))
    o0, o1, o2, o3 = pl.pallas_call(
        _topk_body,
        grid=(_B // _R,),
        in_specs=[
            pl.BlockSpec(memory_space=pltpu.SMEM),
            row_spec, row_spec, row_spec, row_spec,
        ],
        out_specs=[out_spec, out_spec, out_spec, out_spec],
        out_shape=[jax.ShapeDtypeStruct((_B, _K), jnp.float32)] * 4,
        scratch_shapes=[pltpu.VMEM((_R, _N), jnp.float32)],
    )(pref, x0, x1, x2, sols_mask)
    return jnp.stack([o0, o1, o2, o3], axis=-1)


def kernel(sols, sols_mask, pref, k):
    del k
    return _run(sols, sols_mask, pref)
